# Initial kernel scaffold; baseline (speedup 1.0000x reference)
import functools

import jax
import jax.numpy as jnp
from jax import lax
from jax.experimental import pallas as pl
from jax.experimental.pallas import tpu as pltpu
from jax.experimental.pallas import tpu_sc as plsc

VOCAB = 1_000_000
EMBED_DIM = 32
BATCH = 4096
SEQ = 200
LANES = 16

NUM_CORES = 2
NUM_SUBCORES = 16
NW = NUM_CORES * NUM_SUBCORES
BPW = BATCH // NW
NSUB = 2
CHUNK = SEQ // NSUB
RUNROLL = 4


def _sc_classifier(ids_r, embedding, fc_w, fc_b):
    mesh = plsc.VectorSubcoreMesh(core_axis_name="c", subcore_axis_name="s")

    @functools.partial(
        pl.kernel,
        mesh=mesh,
        out_type=jax.ShapeDtypeStruct((BATCH,), jnp.float32),
        scratch_types=[
            pltpu.VMEM((BPW, NSUB, CHUNK), jnp.int32),
            pltpu.VMEM((NSUB, CHUNK, EMBED_DIM), jnp.float32),
            pltpu.VMEM((BPW,), jnp.float32),
            pltpu.VMEM((1, EMBED_DIM), jnp.float32),
            pltpu.VMEM((1,), jnp.float32),
            pltpu.SemaphoreType.DMA,
        ],
    )
    def k(ids_hbm, emb_hbm, fcw_hbm, fcb_hbm, out_hbm,
          idx_v, rows_v, out_v, w_v, b_v, sem):
        wid = lax.axis_index("s") * NUM_CORES + lax.axis_index("c")
        pltpu.sync_copy(ids_hbm.at[wid], idx_v)
        pltpu.sync_copy(fcw_hbm, w_v)
        pltpu.sync_copy(fcb_hbm, b_v)
        w_lo = w_v[0, pl.ds(0, LANES)]
        w_hi = w_v[0, pl.ds(LANES, LANES)]
        bias_vec = jnp.full((LANES,), b_v[0], jnp.float32) * (1.0 / LANES)
        inv_seq = 1.0 / SEQ

        def row_body(b, _):
            copies = [
                pltpu.async_copy(emb_hbm.at[idx_v.at[b, j]], rows_v.at[j], sem)
                for j in range(NSUB)
            ]
            for c in copies:
                c.wait()

            zero = jnp.zeros((LANES,), jnp.float32)
            accs = (zero,) * (2 * RUNROLL)

            def red_body(r, accs):
                a = list(accs)
                for u in range(RUNROLL):
                    row = r * RUNROLL + u
                    for j in range(NSUB):
                        a[u] = a[u] + rows_v[j, row, pl.ds(0, LANES)]
                        a[RUNROLL + u] = (
                            a[RUNROLL + u]
                            + rows_v[j, row, pl.ds(LANES, LANES)]
                        )
                return tuple(a)

            accs = lax.fori_loop(0, CHUNK // RUNROLL, red_body, accs)
            acc_lo = (accs[0] + accs[1]) + (accs[2] + accs[3])
            acc_hi = (accs[4] + accs[5]) + (accs[6] + accs[7])
            t = (acc_lo * w_lo + acc_hi * w_hi) * inv_seq + bias_vec
            out_v[b] = jnp.sum(t)
            return 0

        lax.fori_loop(0, BPW, row_body, 0)
        pltpu.sync_copy(out_v, out_hbm.at[pl.ds(wid * BPW, BPW)])

    return k(ids_r, embedding, fc_w, fc_b)


def kernel(input_ids, embedding, fc_w, fc_b):
    ids_r = input_ids.reshape(NW, BPW, NSUB, CHUNK)
    out = _sc_classifier(ids_r, embedding, fc_w, fc_b)
    return out.reshape(BATCH, 1)


# speedup vs baseline: 2.0663x; 2.0663x over previous
import functools

import jax
import jax.numpy as jnp
from jax import lax
from jax.experimental import pallas as pl
from jax.experimental.pallas import tpu as pltpu
from jax.experimental.pallas import tpu_sc as plsc

VOCAB = 1_000_000
EMBED_DIM = 32
BATCH = 4096
SEQ = 200
LANES = 16

NUM_CORES = 2
NUM_SUBCORES = 16
NW = NUM_CORES * NUM_SUBCORES
BPW = BATCH // NW
NSUB = 2
CHUNK = SEQ // NSUB
RUNROLL = 4


def _sc_classifier(ids_r, embedding, fc_w, fc_b):
    mesh = plsc.VectorSubcoreMesh(core_axis_name="c", subcore_axis_name="s")

    @functools.partial(
        pl.kernel,
        mesh=mesh,
        out_type=jax.ShapeDtypeStruct((NW, BPW // LANES, LANES), jnp.float32),
        scratch_types=[
            pltpu.VMEM((BPW, NSUB, CHUNK), jnp.int32),
            pltpu.VMEM((CHUNK, EMBED_DIM), jnp.float32),
            pltpu.VMEM((CHUNK, EMBED_DIM), jnp.float32),
            pltpu.VMEM((BPW // LANES, LANES), jnp.float32),
            pltpu.VMEM((1, EMBED_DIM), jnp.float32),
            pltpu.VMEM((LANES,), jnp.float32),
            pltpu.SemaphoreType.DMA,
        ],
        compiler_params=pltpu.CompilerParams(
            needs_layout_passes=False, use_tc_tiling_on_sc=False),
    )
    def k(ids_hbm, emb_hbm, fcw_hbm, fcb_hbm, out_hbm,
          idx_v, rows0_v, rows1_v, out_v, w_v, b_v, sem):
        wid = lax.axis_index("s") * NUM_CORES + lax.axis_index("c")
        pltpu.sync_copy(ids_hbm.at[wid], idx_v)
        pltpu.sync_copy(fcw_hbm, w_v)
        pltpu.sync_copy(fcb_hbm, b_v.at[pl.ds(0, 1)])
        w_lo = w_v[0, pl.ds(0, LANES)]
        w_hi = w_v[0, pl.ds(LANES, LANES)]
        b_vec_raw = b_v[pl.ds(0, LANES)]
        bias_vec = jnp.full((LANES,), b_vec_raw[0], jnp.float32) * (1.0 / LANES)
        inv_seq = 1.0 / SEQ
        lane_ids = lax.iota(jnp.int32, LANES)

        def row_body(b, res):
            rows_refs = (rows0_v, rows1_v)
            copies = [
                pltpu.async_copy(emb_hbm.at[idx_v.at[b, j]], rows_refs[j], sem)
                for j in range(NSUB)
            ]
            for c in copies:
                c.wait()

            zero = jnp.zeros((LANES,), jnp.float32)
            accs = (zero,) * (2 * RUNROLL)

            def red_body(r, accs):
                a = list(accs)
                for u in range(RUNROLL):
                    row = r * RUNROLL + u
                    for j in range(NSUB):
                        rr = rows_refs[j]
                        a[u] = a[u] + rr[row, pl.ds(0, LANES)]
                        a[RUNROLL + u] = (
                            a[RUNROLL + u] + rr[row, pl.ds(LANES, LANES)]
                        )
                return tuple(a)

            accs = lax.fori_loop(0, CHUNK // RUNROLL, red_body, accs)
            acc_lo = (accs[0] + accs[1]) + (accs[2] + accs[3])
            acc_hi = (accs[4] + accs[5]) + (accs[6] + accs[7])
            t = (acc_lo * w_lo + acc_hi * w_hi) * inv_seq + bias_vec
            s = jnp.sum(t)
            res = jnp.where(lane_ids == b % LANES,
                            jnp.full((LANES,), s, jnp.float32), res)

            @pl.when(b % LANES == LANES - 1)
            def _():
                out_v[b // LANES] = res

            return res

        lax.fori_loop(0, BPW, row_body, jnp.zeros((LANES,), jnp.float32))
        pltpu.sync_copy(out_v, out_hbm.at[wid])

    return k(ids_r, embedding, fc_w, fc_b)


def kernel(input_ids, embedding, fc_w, fc_b):
    ids_r = input_ids.reshape(NW, BPW, NSUB, CHUNK)
    out = _sc_classifier(ids_r, embedding, fc_w, fc_b)
    return out.reshape(BATCH, 1)
